# Initial kernel scaffold; baseline (speedup 1.0000x reference)
import jax
import jax.numpy as jnp
from jax.experimental import pallas as pl


def kernel(x, delta_p, delta_t):
    raise NotImplementedError("write your pallas kernel here")


# speedup vs baseline: 1.8112x; 1.8112x over previous
import functools

import jax
import jax.numpy as jnp
from jax import lax
from jax.experimental import pallas as pl
from jax.experimental.pallas import tpu as pltpu
from jax.experimental.pallas import tpu_sc as plsc

B, C, H, W = 4, 96, 224, 224
HW = H * W
NW = 32
CH = 112
TPW = B * HW // NW
NCH = TPW // CH
GRP = CH // 16

_mesh = plsc.VectorSubcoreMesh(core_axis_name="c", subcore_axis_name="s")


@functools.partial(
    pl.kernel,
    mesh=_mesh,
    compiler_params=pltpu.CompilerParams(use_tc_tiling_on_sc=False),
    out_type=jax.ShapeDtypeStruct((B * HW, C), jnp.float32),
    scratch_types=[
        pltpu.VMEM((CH,), jnp.int32),
        pltpu.VMEM((CH,), jnp.int32),
        pltpu.VMEM((CH,), jnp.int32),
        pltpu.VMEM((CH,), jnp.int32),
        pltpu.VMEM((CH,), jnp.int32),
        pltpu.VMEM((CH,), jnp.float32),
        pltpu.VMEM((CH,), jnp.float32),
        pltpu.VMEM((CH,), jnp.float32),
        pltpu.VMEM((CH,), jnp.float32),
        pltpu.VMEM((CH, C), jnp.float32),
        pltpu.VMEM((CH, C), jnp.float32),
        pltpu.VMEM((CH, C), jnp.float32),
        pltpu.VMEM((CH, C), jnp.float32),
        pltpu.VMEM((CH, C), jnp.float32),
        pltpu.SemaphoreType.DMA,
    ],
)
def _sc_gather(xt_hbm, n0_hbm, n1_hbm, n2_hbm, n3_hbm,
               v0_hbm, v1_hbm, v2_hbm, v3_hbm, sidx_hbm, out_hbm,
               sidx_v, i0, i1, i2, i3, w0, w1, w2, w3,
               nb0, nb1, nb2, nb3, ob, sem):
    wid = lax.axis_index("s") * 2 + lax.axis_index("c")
    wbase = wid * TPW

    def chunk_body(ci, carry):
        gbase = wbase + ci * CH
        pltpu.sync_copy(sidx_hbm.at[pl.ds(gbase, CH)], sidx_v)
        cps = [
            pltpu.async_copy(n0_hbm.at[sidx_v], i0, sem),
            pltpu.async_copy(n1_hbm.at[sidx_v], i1, sem),
            pltpu.async_copy(n2_hbm.at[sidx_v], i2, sem),
            pltpu.async_copy(n3_hbm.at[sidx_v], i3, sem),
            pltpu.async_copy(v0_hbm.at[sidx_v], w0, sem),
            pltpu.async_copy(v1_hbm.at[sidx_v], w1, sem),
            pltpu.async_copy(v2_hbm.at[sidx_v], w2, sem),
            pltpu.async_copy(v3_hbm.at[sidx_v], w3, sem),
        ]
        for cp in cps:
            cp.wait()

        cps = [
            pltpu.async_copy(xt_hbm.at[i0], nb0, sem),
            pltpu.async_copy(xt_hbm.at[i1], nb1, sem),
            pltpu.async_copy(xt_hbm.at[i2], nb2, sem),
            pltpu.async_copy(xt_hbm.at[i3], nb3, sem),
        ]
        for cp in cps:
            cp.wait()

        def tok_body(g, c2):
            base = g * 16
            aw0 = w0[pl.ds(base, 16)]
            aw1 = w1[pl.ds(base, 16)]
            aw2 = w2[pl.ds(base, 16)]
            aw3 = w3[pl.ds(base, 16)]
            for l in range(16):
                t = base + l
                a0 = aw0[l]
                a1 = aw1[l]
                a2 = aw2[l]
                a3 = aw3[l]
                for v in range(C // 16):
                    s = pl.ds(v * 16, 16)
                    ob[t, s] = nb0[t, s] * a0 + nb1[t, s] * a1 \
                        + nb2[t, s] * a2 + nb3[t, s] * a3
            return c2

        lax.fori_loop(0, GRP, tok_body, 0)
        pltpu.sync_copy(ob, out_hbm.at[pl.ds(gbase, CH)])
        return carry

    lax.fori_loop(0, NCH, chunk_body, 0)


def kernel(x, delta_p, delta_t):
    b, c, h, w = x.shape
    hw = h * w

    gyy, gxx = jnp.meshgrid(jnp.linspace(-1.0, 1.0, h),
                            jnp.linspace(-1.0, 1.0, w), indexing="ij")
    gx = gxx[None] + delta_p[:, 0]
    gy = gyy[None] + delta_p[:, 1]
    ix = ((gx + 1.0) * w - 1.0) / 2.0
    iy = ((gy + 1.0) * h - 1.0) / 2.0
    ix0 = jnp.floor(ix)
    iy0 = jnp.floor(iy)
    ix1 = ix0 + 1.0
    iy1 = iy0 + 1.0
    wx1 = ix - ix0
    wy1 = iy - iy0
    wx0 = 1.0 - wx1
    wy0 = 1.0 - wy1

    boff = (jnp.arange(b, dtype=jnp.int32) * hw)[:, None, None]

    def corner(ixq, iyq, wt):
        valid = (ixq >= 0.0) & (ixq <= w - 1.0) & (iyq >= 0.0) & (iyq <= h - 1.0)
        ixc = jnp.clip(ixq, 0.0, w - 1.0).astype(jnp.int32)
        iyc = jnp.clip(iyq, 0.0, h - 1.0).astype(jnp.int32)
        n = iyc * w + ixc + boff
        wv = jnp.where(valid, wt, 0.0)
        return n.reshape(b * hw), wv.reshape(b * hw)

    n00, w00 = corner(ix0, iy0, wx0 * wy0)
    n01, w01 = corner(ix1, iy0, wx1 * wy0)
    n10, w10 = corner(ix0, iy1, wx0 * wy1)
    n11, w11 = corner(ix1, iy1, wx1 * wy1)

    ref_idx = (jnp.arange(hw, dtype=jnp.float32).reshape(1, 1, h, w)
               / (hw - 1) * 2.0 - 1.0)
    keys = (ref_idx + delta_t).reshape(b, hw)
    sidx = jnp.argsort(keys, axis=1).astype(jnp.int32)
    sidx = (sidx + (jnp.arange(b, dtype=jnp.int32) * hw)[:, None]).reshape(b * hw)

    xt = jnp.transpose(x.reshape(b, c, hw), (0, 2, 1)).reshape(b * hw, c)

    out = _sc_gather(xt, n00, n01, n10, n11, w00, w01, w10, w11, sidx)
    return out.reshape(b, hw, c)
